# baseline (speedup 1.0000x reference)
import functools
import jax
import jax.numpy as jnp
from jax import lax
from jax.experimental import pallas as pl
from jax.experimental.pallas import tpu as pltpu
from jax.experimental.pallas import tpu_sc as plsc

D = 64
SCALE = 8.0
NC, NS = 2, 16
NW = NC * NS
SEQ = 200
BATCH = 4096
B = BATCH * SEQ
V = 1000000
BPW = B // NW
NCHW = BPW // 128
TD, DR = D // 8, 8
NTB = BATCH // 128
NVC = 7813
KA = 248

_mesh = plsc.VectorSubcoreMesh(
    core_axis_name="c", subcore_axis_name="s", num_cores=NC, num_subcores=NS
)
_params = pltpu.CompilerParams(needs_layout_passes=False)


@functools.partial(
    pl.kernel,
    out_type=jax.ShapeDtypeStruct((V // 2, 128), jnp.float32),
    mesh=_mesh,
    scratch_types=[
        [pltpu.VMEM((D, 128), jnp.float32) for _ in range(4)],
        [pltpu.VMEM((D, 128), jnp.float32) for _ in range(2)],
        [pltpu.SemaphoreType.DMA for _ in range(4)],
        [pltpu.SemaphoreType.DMA for _ in range(2)],
    ],
    compiler_params=_params,
)
def _convert(lutt_hbm, out_hbm, tin, tout, isem, osem):
    wid = lax.axis_index("s") * NC + lax.axis_index("c")
    iota = lax.iota(jnp.int32, 16)

    def fire_in(c, r):
        vc = pl.multiple_of(c * 128, 128)
        pltpu.async_copy(lutt_hbm.at[:, pl.ds(vc, 128)], tin[r], isem[r])

    def drain_in(c, r):
        vc = pl.multiple_of(c * 128, 128)
        pltpu.make_async_copy(
            lutt_hbm.at[:, pl.ds(vc, 128)], tin[r], isem[r]
        ).wait()

    def transform(rin, rout):
        @pl.loop(0, D // 2)
        def _rows(pp):
            vs = []
            for psub in range(2):
                for h in range(2):
                    col = jnp.full((16,), h, jnp.int32) + (pp * 4 + psub * 2)
                    for cs in range(4):
                        v = plsc.load_gather(tin[rin], [iota + cs * 16, col])
                        vs.append((psub, h, cs, v * SCALE))
            for psub, h, cs, v in vs:
                tout[rout][pp * 2 + psub, pl.ds(h * 64 + cs * 16, 16)] = v

    def _out_copy(c, r, rows_n):
        return pltpu.make_async_copy(
            tout[r].at[pl.ds(0, rows_n)],
            out_hbm.at[pl.ds(pl.multiple_of(c * 64, 8), rows_n)],
            osem[r],
        )

    def fire_out(c, r):
        @pl.when(c < NVC - 1)
        def _():
            _out_copy(c, r, D).start()

        @pl.when(c == NVC - 1)
        def _():
            _out_copy(c, r, D // 2).start()

    def drain_out(c, r):
        @pl.when(c < NVC - 1)
        def _():
            _out_copy(c, r, D).wait()

        @pl.when(c == NVC - 1)
        def _():
            _out_copy(c, r, D // 2).wait()

    for kp in range(3):
        fire_in(wid + 32 * kp, kp)

    @pl.loop(0, KA // 4)
    def _step(j):
        for rr in range(4):
            k = 4 * j + rr
            c = wid + 32 * k
            rin = rr
            rout = rr % 2

            @pl.when(c < NVC)
            def _():
                drain_in(c, rin)

            @pl.when(c + 96 < NVC)
            def _():
                fire_in(c + 96, (rr + 3) % 4)

            @pl.when(jnp.logical_and(k >= 2, c - 64 < NVC))
            def _():
                drain_out(c - 64, rout)

            @pl.when(c < NVC)
            def _():
                transform(rin, rout)
                fire_out(c, rout)

    for k in (KA - 2, KA - 1):
        ce = wid + 32 * k

        @pl.when(ce < NVC)
        def _():
            drain_out(ce, k % 2)


@functools.partial(
    pl.kernel,
    out_type=jax.ShapeDtypeStruct((SEQ * D * BATCH,), jnp.float32),
    mesh=_mesh,
    scratch_types=[
        pltpu.VMEM((BPW,), jnp.int32),
        [pltpu.VMEM((128,), jnp.int32) for _ in range(4)],
        [pltpu.VMEM((128, 128), jnp.float32) for _ in range(4)],
        [pltpu.VMEM((TD * DR * 128,), jnp.float32) for _ in range(2)],
        [pltpu.SemaphoreType.DMA for _ in range(4)],
        [pltpu.SemaphoreType.DMA for _ in range(2)],
    ],
    compiler_params=_params,
)
def _gather(x_hbm, lutp_hbm, out_hbm, idx_v, line, rows, stage, gsem, osem):
    wid = lax.axis_index("s") * NC + lax.axis_index("c")
    iota = lax.iota(jnp.int32, 16)

    pltpu.sync_copy(x_hbm.at[wid], idx_v)

    def compute_lines(k, r):
        for i in range(8):
            line[r][pl.ds(i * 16, 16)] = (
                idx_v[pl.ds(k * 128 + i * 16, 16)] >> 1
            )

    def fire_gather(r):
        pltpu.async_copy(lutp_hbm.at[line[r]], rows[r], gsem[r])

    def drain_gather(r):
        pltpu.make_async_copy(lutp_hbm.at[line[r]], rows[r], gsem[r]).wait()

    def transform(k, rin, rout):
        @pl.loop(0, 8)
        def _bblock(bb):
            b0 = bb * 16
            halves = (idx_v[pl.ds(k * 128 + b0, 16)] & 1) << 6
            row_ids = iota + b0
            for og in range(8):
                colg = halves + og * 8
                vs = []
                for oo in range(8):
                    o = og * 8 + oo
                    v = plsc.load_gather(rows[rin], [row_ids, colg + oo])
                    vs.append((o, v))
                for o, v in vs:
                    doff = (o >> 3) * 1024 + (o & 7) * 128
                    stage[rout][pl.ds(doff + b0, 16)] = v

    def _write_copies(c, r):
        s = c >> 5
        tb = c & 31
        base = s * (D * BATCH) + tb * 1024
        return [
            pltpu.make_async_copy(
                stage[r].at[pl.ds(td * 1024, 1024)],
                out_hbm.at[
                    pl.ds(pl.multiple_of(base + td * (DR * BATCH), 1024), 1024)
                ],
                osem[r],
            )
            for td in range(TD)
        ]

    def fire_write(c, r):
        for cp in _write_copies(c, r):
            cp.start()

    def drain_write(c, r):
        for cp in _write_copies(c, r):
            cp.wait()

    c0 = wid * NCHW
    for kp in range(3):
        compute_lines(kp, kp)
        fire_gather(kp)

    @pl.loop(0, NCHW // 4)
    def _step(j):
        for rr in range(4):
            k = 4 * j + rr
            rout = rr % 2
            drain_gather(rr)

            @pl.when(k + 3 < NCHW)
            def _():
                compute_lines(k + 3, (rr + 3) % 4)
                fire_gather((rr + 3) % 4)

            @pl.when(k >= 2)
            def _():
                drain_write(c0 + k - 2, rout)

            transform(k, rr, rout)
            fire_write(c0 + k, rout)

    drain_write(c0 + NCHW - 2, 0)
    drain_write(c0 + NCHW - 1, 1)


def kernel(x, lut):
    lutp = _convert(lut.T)
    xf = x.T.reshape(NW, BPW).astype(jnp.int32)
    flat = _gather(xf, lutp)
    out5 = flat.reshape(SEQ, TD, NTB, DR, 128)
    return out5.transpose(2, 4, 0, 1, 3).reshape(BATCH, SEQ, D)


# speedup vs baseline: 1.4650x; 1.4650x over previous
import functools
import jax
import jax.numpy as jnp
from jax import lax
from jax.experimental import pallas as pl
from jax.experimental.pallas import tpu as pltpu
from jax.experimental.pallas import tpu_sc as plsc

D = 64
SCALE = 8.0
NC, NS = 2, 16
NW = NC * NS
B = 4096 * 200
BPW = B // NW
IDX_MINOR = 128
NIDXROW = BPW // IDX_MINOR
CHUNK = 256
GPC = CHUNK // IDX_MINOR
NCHUNK = BPW // CHUNK
NBUF = 4

_mesh = plsc.VectorSubcoreMesh(
    core_axis_name="c", subcore_axis_name="s", num_cores=NC, num_subcores=NS
)


@functools.partial(
    pl.kernel,
    out_type=jax.ShapeDtypeStruct((B, D), jnp.float32),
    mesh=_mesh,
    scratch_types=[
        pltpu.VMEM((NIDXROW, IDX_MINOR), jnp.int32),
        [pltpu.VMEM((CHUNK, D), jnp.float32) for _ in range(NBUF)],
        [pltpu.SemaphoreType.DMA for _ in range(NBUF)],
        [pltpu.SemaphoreType.DMA for _ in range(NBUF)],
    ],
    compiler_params=pltpu.CompilerParams(use_tc_tiling_on_sc=False),
)
def _emb_lookup(x_hbm, lut_hbm, out_hbm, idx_v, rows, gsem, osem):
    wid = lax.axis_index("s") * NC + lax.axis_index("c")
    base = wid * BPW

    pltpu.sync_copy(x_hbm.at[wid], idx_v)

    def fire_gathers(g, r):
        for j in range(GPC):
            pltpu.async_copy(
                lut_hbm.at[idx_v.at[g * GPC + j]],
                rows[r].at[pl.ds(j * IDX_MINOR, IDX_MINOR)],
                gsem[r],
            )

    def drain_gathers(g, r):
        for j in range(GPC):
            pltpu.make_async_copy(
                lut_hbm.at[idx_v.at[g * GPC + j]],
                rows[r].at[pl.ds(j * IDX_MINOR, IDX_MINOR)],
                gsem[r],
            ).wait()

    def scale(r):
        @pl.loop(0, CHUNK, unroll=8)
        def _scale(row):
            for c in range(D // 16):
                rows[r][row, pl.ds(c * 16, 16)] = (
                    rows[r][row, pl.ds(c * 16, 16)] * SCALE
                )

    def fire_write(g, r):
        pltpu.async_copy(
            rows[r], out_hbm.at[pl.ds(base + g * CHUNK, CHUNK)], osem[r]
        )

    def drain_write(g, r):
        pltpu.make_async_copy(
            rows[r], out_hbm.at[pl.ds(base + g * CHUNK, CHUNK)], osem[r]
        ).wait()

    for r in range(NBUF - 1):
        fire_gathers(r, r)

    @pl.loop(0, NCHUNK // NBUF)
    def _step(k):
        for r in range(NBUF):
            g = k * NBUF + r
            drain_gathers(g, r)
            scale(r)

            rn = (r + NBUF - 1) % NBUF
            gn = g + NBUF - 1

            @pl.when(gn < NCHUNK)
            def _():
                @pl.when(g >= 1)
                def _():
                    drain_write(gn - NBUF, rn)

                fire_gathers(gn, rn)

            fire_write(g, r)

    for r in range(NBUF):
        g = NCHUNK - NBUF + r
        drain_write(g, g % NBUF)


def kernel(x, lut):
    xf = x.reshape(NW, NIDXROW, IDX_MINOR).astype(jnp.int32)
    out = _emb_lookup(xf, lut)
    return out.reshape(x.shape[0], x.shape[1], D)
